# Initial kernel scaffold; baseline (speedup 1.0000x reference)
import jax
import jax.numpy as jnp
from jax.experimental import pallas as pl


def kernel(edge_index, rel_h, ent_emb, user_emb, latent_emb, rel_emb, disen_att, interact_user_idx, interact_ent_idx, interact_vals):
    raise NotImplementedError("write your pallas kernel here")


# speedup vs baseline: 2.8010x; 2.8010x over previous
import functools

import jax
import jax.numpy as jnp
from jax import lax
from jax.experimental import pallas as pl
from jax.experimental.pallas import tpu as pltpu
from jax.experimental.pallas import tpu_sc as plsc

N = 10000
E = 320000
D = 128
U = 20000
NNZ = 400000
NF = 4
R = 16

C = 128
NS = 16
NC = 2
LANES = 16

NPAD = 10240
RPT = NPAD // NS
RBLKS = RPT // C

E_CHUNKS = E // C
E_SC = E_CHUNKS // NC
E_TRIPS = (E_SC + NS - 1) // NS
NNZ_CHUNKS = NNZ // C
NNZ_TRIPS = (NNZ_CHUNKS + NS - 1) // NS

U_HALF = U // NC
DUMP = U_HALF
NDEG = 10112


@functools.cache
def _get_mesh():
    return plsc.VectorSubcoreMesh(core_axis_name="c", subcore_axis_name="s")


def _fill_f32(buf, rows, width, value):
    v = jnp.full((LANES,), value, jnp.float32)

    def body(i, _):
        for q in range(width // LANES):
            buf[i, pl.ds(q * LANES, LANES)] = v
        return 0

    lax.fori_loop(0, rows, body, 0)


def _fill_bf16(buf, rows, width, value):
    v = jnp.full((2, LANES), value, jnp.bfloat16)

    def body(i, _):
        r = pl.multiple_of(i * 2, 2)
        for q in range(width // LANES):
            buf[pl.ds(r, 2), pl.ds(q * LANES, LANES)] = v
        return 0

    lax.fori_loop(0, rows // 2, body, 0)


def _neigh_body(src_hbm, dst_hbm, rel_hbm, ent_hbm, acc_out,
                src_v, dst_v, ent_rows, rel_rows, acc_sh, sem):
    c = lax.axis_index("c")
    s = lax.axis_index("s")
    r0 = s * RPT

    _fill_f32(ent_rows, C, D, 0.0)
    for k in range(RBLKS):
        pltpu.sync_copy(ent_rows, acc_sh.at[pl.ds(r0 + k * C, C)])
    plsc.subcore_barrier()

    def chunk_body(j, _):
        local = s + j * NS

        @pl.when(local < E_SC)
        def _():
            e0 = (c * E_SC + local) * C
            pltpu.sync_copy(src_hbm.at[pl.ds(e0, C)], src_v)
            pltpu.sync_copy(dst_hbm.at[pl.ds(e0, C)], dst_v)
            pltpu.async_copy(ent_hbm.at[src_v], ent_rows, sem).wait()
            pltpu.sync_copy(rel_hbm.at[pl.ds(e0, C)], rel_rows)

            def mul_body(i, _):
                for q in range(D // LANES):
                    sl = pl.ds(q * LANES, LANES)
                    ent_rows[i, sl] = ent_rows[i, sl] * rel_rows[i, sl]
                return 0

            lax.fori_loop(0, C, mul_body, 0)
            pltpu.sync_copy(ent_rows, acc_sh.at[dst_v], add=True)

        return 0

    lax.fori_loop(0, E_TRIPS, chunk_body, 0)
    plsc.subcore_barrier()

    for k in range(RBLKS):
        blk = (c * NS + s) * RBLKS + k
        pltpu.sync_copy(acc_sh.at[pl.ds(r0 + k * C, C)], ent_rows)
        pltpu.sync_copy(ent_rows, acc_out.at[blk])


@functools.cache
def _neigh_sc():
    return pl.kernel(
        _neigh_body,
        out_type=jax.ShapeDtypeStruct((NC * NS * RBLKS, C, D), jnp.float32),
        mesh=_get_mesh(),
        scratch_types=[
            pltpu.VMEM((C,), jnp.int32),
            pltpu.VMEM((C,), jnp.int32),
            pltpu.VMEM((C, D), jnp.float32),
            pltpu.VMEM((C, D), jnp.float32),
            pltpu.VMEM_SHARED((NPAD, D), jnp.float32),
            pltpu.SemaphoreType.DMA,
        ],
    )


def _user_body(uidx_hbm, eidx_hbm, vals_hbm, dst_hbm, ent_hbm, out_hbm,
               deg_out, uidx_v, eidx_v, vals_v, ent_rows, deg_local,
               acc_sh, sem):
    c = lax.axis_index("c")
    s = lax.axis_index("s")
    c_u = c * U_HALF
    r0 = s * RPT

    _fill_f32(ent_rows, C, D, 0.0)
    for k in range(RBLKS):
        pltpu.sync_copy(ent_rows, acc_sh.at[pl.ds(r0 + k * C, C)])

    zv = jnp.zeros((LANES,), jnp.float32)

    def zdeg_body(i, _):
        deg_local[pl.ds(i * LANES, LANES)] = zv
        return 0

    lax.fori_loop(0, NDEG // LANES, zdeg_body, 0)
    onehot = jnp.where(lax.iota(jnp.int32, LANES) == 0, 1.0, 0.0
                       ).astype(jnp.float32)
    plsc.subcore_barrier()

    def deg_chunk(j, _):
        local = s + j * NS

        @pl.when(local < E_SC)
        def _():
            e0 = (c * E_SC + local) * C
            pltpu.sync_copy(dst_hbm.at[pl.ds(e0, C)], eidx_v)

            def deg_body(g, _):
                vidx = eidx_v[pl.ds(g * LANES, LANES)]
                for r in range(LANES):
                    dyn = pl.ds(vidx[r], LANES)
                    deg_local[dyn] = deg_local[dyn] + onehot
                return 0

            lax.fori_loop(0, C // LANES, deg_body, 0)

        return 0

    lax.fori_loop(0, E_TRIPS, deg_chunk, 0)
    pltpu.sync_copy(deg_local, deg_out.at[c * NS + s])

    dump_v = jnp.full((LANES,), DUMP, jnp.int32)

    def chunk_body(j, _):
        chunk = s + j * NS

        @pl.when(chunk < NNZ_CHUNKS)
        def _():
            e0 = chunk * C
            pltpu.sync_copy(uidx_hbm.at[pl.ds(e0, C)], uidx_v)
            pltpu.sync_copy(eidx_hbm.at[pl.ds(e0, C)], eidx_v)
            pltpu.sync_copy(vals_hbm.at[pl.ds(e0, C)], vals_v)
            for q in range(C // LANES):
                sl = pl.ds(q * LANES, LANES)
                ul = uidx_v[sl] - c_u
                sel = (ul >= 0) & (ul < U_HALF)
                uidx_v[sl] = jnp.where(sel, ul, dump_v)
            pltpu.async_copy(ent_hbm.at[eidx_v], ent_rows, sem).wait()

            def mul_body(g, _):
                vv = vals_v[pl.ds(g * LANES, LANES)]
                for r in range(LANES):
                    sv = vv[r]
                    i = g * LANES + r
                    for q in range(D // LANES):
                        sl = pl.ds(q * LANES, LANES)
                        ent_rows[i, sl] = ent_rows[i, sl] * sv
                return 0

            lax.fori_loop(0, C // LANES, mul_body, 0)
            pltpu.sync_copy(ent_rows, acc_sh.at[uidx_v], add=True)

        return 0

    lax.fori_loop(0, NNZ_TRIPS, chunk_body, 0)
    plsc.subcore_barrier()

    for k in range(RBLKS):
        blk = (c * NS + s) * RBLKS + k
        pltpu.sync_copy(acc_sh.at[pl.ds(r0 + k * C, C)], ent_rows)
        pltpu.sync_copy(ent_rows, out_hbm.at[blk])


@functools.cache
def _user_sc():
    return pl.kernel(
        _user_body,
        out_type=(
            jax.ShapeDtypeStruct((NC * NS * RBLKS, C, D), jnp.float32),
            jax.ShapeDtypeStruct((NC * NS, NDEG), jnp.float32),
        ),
        mesh=_get_mesh(),
        scratch_types=[
            pltpu.VMEM((C,), jnp.int32),
            pltpu.VMEM((C,), jnp.int32),
            pltpu.VMEM((C,), jnp.float32),
            pltpu.VMEM((C, D), jnp.float32),
            pltpu.VMEM((NDEG,), jnp.float32),
            pltpu.VMEM_SHARED((NPAD, D), jnp.float32),
            pltpu.SemaphoreType.DMA,
        ],
    )


BN = 2000
BU = 2000


def _merge_body(p_ref, d_ref, out_ref):
    d = jnp.sum(d_ref[...], axis=1)[:, None]
    out_ref[...] = (p_ref[0] + p_ref[1]) / jnp.maximum(d, 1.0)


def _merge_tc(p, deg_t):
    return pl.pallas_call(
        _merge_body,
        grid=(N // BN,),
        in_specs=[
            pl.BlockSpec((2, BN, D), lambda i: (0, i, 0)),
            pl.BlockSpec((BN, NC * NS), lambda i: (i, 0)),
        ],
        out_specs=pl.BlockSpec((BN, D), lambda i: (i, 0)),
        out_shape=jax.ShapeDtypeStruct((N, D), jnp.float32),
    )(p, deg_t)


def _attn_body(ua_ref, ue_ref, lat_ref, datt_ref, rel_ref, out_ref):
    datt = datt_ref[...]
    sm = jax.nn.softmax(datt, axis=-1)
    de = jnp.dot(sm, rel_ref[...], preferred_element_type=jnp.float32,
                 precision=lax.Precision.HIGHEST)
    score = lax.dot_general(ue_ref[...], lat_ref[...],
                            (((1,), (1,)), ((), ())),
                            preferred_element_type=jnp.float32,
                            precision=lax.Precision.HIGHEST)
    w = jnp.dot(score, de, preferred_element_type=jnp.float32,
                precision=lax.Precision.HIGHEST)
    out_ref[...] = ua_ref[0] * (w + 1.0)


def _attn_tc(uh, user_emb, latent_emb, rel_emb, disen_att):
    nb = U_HALF // BU
    return pl.pallas_call(
        _attn_body,
        grid=(U // BU,),
        in_specs=[
            pl.BlockSpec((1, BU, D), lambda i: (i // nb, i % nb, 0)),
            pl.BlockSpec((BU, D), lambda i: (i, 0)),
            pl.BlockSpec((NF, D), lambda i: (0, 0)),
            pl.BlockSpec((NF, R), lambda i: (0, 0)),
            pl.BlockSpec((R, D), lambda i: (0, 0)),
        ],
        out_specs=pl.BlockSpec((BU, D), lambda i: (i, 0)),
        out_shape=jax.ShapeDtypeStruct((U, D), jnp.float32),
    )(uh, user_emb, latent_emb, disen_att, rel_emb)


def kernel(edge_index, rel_h, ent_emb, user_emb, latent_emb, rel_emb,
           disen_att, interact_user_idx, interact_ent_idx, interact_vals):
    ei = edge_index.astype(jnp.int32)
    acc_p = _neigh_sc()(ei[0], ei[1], rel_h, ent_emb)
    uh, deg_p = _user_sc()(interact_user_idx.astype(jnp.int32),
                           interact_ent_idx.astype(jnp.int32),
                           interact_vals, ei[1], ent_emb)
    neigh = _merge_tc(acc_p.reshape(NC, NPAD, D), deg_p[:, :N].T)
    user_out = _attn_tc(uh.reshape(NC, NPAD, D), user_emb, latent_emb,
                        rel_emb, disen_att)
    return neigh, user_out
